# baseline (speedup 1.0000x reference)
import functools

import jax
import jax.numpy as jnp
import numpy as np
from jax import lax
from jax.experimental import pallas as pl
from jax.experimental.pallas import tpu as pltpu
from jax.experimental.pallas import tpu_sc as plsc

D_MODEL = 128
NUM_ACTIONS = 1000
VOCAB = NUM_ACTIONS + 1
SEQ = 201
SEQ_PAD = 208
EPS = 1e-12
SQRT_D = np.sqrt(D_MODEL)

A_TILE = 16
PID_TILE = 256

BATCH = 4096
NUM_WORKERS = 32
ROWS_PW = BATCH // NUM_WORKERS


def _pairtab_kernel(act_ref, pos_ref, w_ref, b_ref, out_ref):
    x = act_ref[...][:, None, :] * SQRT_D + pos_ref[...][None, :, :]
    mean = jnp.mean(x, axis=-1, keepdims=True)
    var = jnp.mean(jnp.square(x - mean), axis=-1, keepdims=True)
    normed = (x - mean) * jax.lax.rsqrt(var + EPS)
    res = normed * w_ref[...][None] + b_ref[...][None]
    out_ref[...] = res.reshape(A_TILE * SEQ_PAD, D_MODEL)


def _pid_kernel(acts_ref, out_ref):
    c = jax.lax.broadcasted_iota(jnp.int32, (PID_TILE, 2 * SEQ), 1)
    s = jnp.where(c < SEQ, c, c - SEQ)
    out_ref[...] = acts_ref[...] * SEQ_PAD + s


NIDX = 8
N_SPLIT = 2
PAIRS_H = BATCH // 2 // NUM_WORKERS // N_SPLIT
CHUNK = 2 * SEQ


def _sc_gather_kernel(tab_hbm, pid_hbm, out_hbm, idx, rows, si, sg, sw):
    wid = lax.axis_index("s") * 2 + lax.axis_index("c")
    pr0 = wid * PAIRS_H

    for k in range(4):
        pltpu.async_copy(pid_hbm.at[pr0 + k], idx[k], si[k])

    @pl.loop(0, PAIRS_H, step=NIDX)
    def _(r):
        for k in range(NIDX):
            rr = r + k
            j = k % 2
            b0 = (pr0 + rr) * 2

            @pl.when(rr >= 2)
            def _():
                pltpu.make_async_copy(
                    rows[j].at[pl.ds(0, SEQ)], out_hbm.at[b0 - 4], sw[j]
                ).wait()
                pltpu.make_async_copy(
                    rows[j].at[pl.ds(SEQ, SEQ)], out_hbm.at[b0 - 3], sw[j]
                ).wait()

            pltpu.make_async_copy(pid_hbm.at[pr0 + rr], idx[k], si[k]).wait()
            pltpu.async_copy(tab_hbm.at[idx[k]], rows[j], sg[j])
            pltpu.make_async_copy(tab_hbm.at[idx[k]], rows[j], sg[j]).wait()

            kf = (k + 4) % NIDX

            @pl.when(rr + 4 < PAIRS_H)
            def _():
                pltpu.async_copy(pid_hbm.at[pr0 + rr + 4], idx[kf], si[kf])

            pltpu.async_copy(
                rows[j].at[pl.ds(0, SEQ)], out_hbm.at[b0], sw[j]
            )
            pltpu.async_copy(
                rows[j].at[pl.ds(SEQ, SEQ)], out_hbm.at[b0 + 1], sw[j]
            )

    for j in range(2):
        rr = PAIRS_H - 2 + j
        b0 = (pr0 + rr) * 2
        pltpu.make_async_copy(
            rows[j].at[pl.ds(0, SEQ)], out_hbm.at[b0], sw[j]
        ).wait()
        pltpu.make_async_copy(
            rows[j].at[pl.ds(SEQ, SEQ)], out_hbm.at[b0 + 1], sw[j]
        ).wait()


def _sc_gather_body(tab_hbm, pid_hbm, out_hbm, *rest):
    idx = list(rest[:NIDX])
    rows = list(rest[NIDX : NIDX + 2])
    si = list(rest[NIDX + 2 : 2 * NIDX + 2])
    sg = list(rest[2 * NIDX + 2 : 2 * NIDX + 4])
    sw = list(rest[2 * NIDX + 4 : 2 * NIDX + 6])
    _sc_gather_kernel(tab_hbm, pid_hbm, out_hbm, idx, rows, si, sg, sw)


@jax.jit
def kernel(actions, att_mask, action_table, pos_table, ln_weight, ln_bias):
    batch = actions.shape[0]
    cls_col = jnp.full((batch, 1), NUM_ACTIONS, dtype=actions.dtype)
    acts = jnp.concatenate([cls_col, actions], axis=1)
    pos_pad = jnp.pad(pos_table, ((0, SEQ_PAD - SEQ), (0, 0)))
    mask = jnp.concatenate(
        [jnp.zeros((batch, 1), dtype=att_mask.dtype), att_mask], axis=1
    )

    tab_flat = pl.pallas_call(
        _pairtab_kernel,
        grid=(pl.cdiv(VOCAB, A_TILE),),
        in_specs=[
            pl.BlockSpec((A_TILE, D_MODEL), lambda i: (i, 0)),
            pl.BlockSpec((SEQ_PAD, D_MODEL), lambda i: (0, 0)),
            pl.BlockSpec((1, D_MODEL), lambda i: (0, 0)),
            pl.BlockSpec((1, D_MODEL), lambda i: (0, 0)),
        ],
        out_specs=pl.BlockSpec((A_TILE * SEQ_PAD, D_MODEL), lambda i: (i, 0)),
        out_shape=jax.ShapeDtypeStruct((VOCAB * SEQ_PAD, D_MODEL), jnp.float32),
    )(
        action_table,
        pos_pad,
        ln_weight.reshape(1, D_MODEL),
        ln_bias.reshape(1, D_MODEL),
    )

    acts2 = acts.reshape(batch // 2, 2 * SEQ)
    pid = pl.pallas_call(
        _pid_kernel,
        grid=(batch // 2 // PID_TILE,),
        in_specs=[pl.BlockSpec((PID_TILE, 2 * SEQ), lambda i: (i, 0))],
        out_specs=pl.BlockSpec((PID_TILE, 2 * SEQ), lambda i: (i, 0)),
        out_shape=jax.ShapeDtypeStruct((batch // 2, 2 * SEQ), jnp.int32),
    )(acts2)

    mesh = plsc.VectorSubcoreMesh(core_axis_name="c", subcore_axis_name="s")
    sc_gather = functools.partial(
        pl.kernel,
        mesh=mesh,
        out_type=jax.ShapeDtypeStruct((batch // N_SPLIT, SEQ, D_MODEL),
                                      jnp.float32),
        scratch_types=(
            [pltpu.VMEM((CHUNK,), jnp.int32)] * NIDX
            + [pltpu.VMEM((CHUNK, D_MODEL), jnp.float32)] * 2
            + [pltpu.SemaphoreType.DMA] * (NIDX + 4)
        ),
    )(_sc_gather_body)
    half_pairs = batch // 2 // N_SPLIT
    halves = [
        sc_gather(tab_flat, pid[i * half_pairs : (i + 1) * half_pairs])
        for i in range(N_SPLIT)
    ]
    out = jnp.concatenate(halves, axis=0)

    return (out, mask)


# speedup vs baseline: 1.3371x; 1.3371x over previous
import functools

import jax
import jax.numpy as jnp
import numpy as np
from jax import lax
from jax.experimental import pallas as pl
from jax.experimental.pallas import tpu as pltpu
from jax.experimental.pallas import tpu_sc as plsc

D_MODEL = 128
NUM_ACTIONS = 1000
VOCAB = NUM_ACTIONS + 1
SEQ = 201
SEQ_PAD = 208
EPS = 1e-12
SQRT_D = np.sqrt(D_MODEL)

A_TILE = 16
PID_TILE = 256

BATCH = 4096
NUM_WORKERS = 32
ROWS_PW = BATCH // NUM_WORKERS


def _pairtab_kernel(act_ref, pos_ref, w_ref, b_ref, out_ref):
    x3 = act_ref[...][:, None, :] * SQRT_D + pos_ref[...][None, :, :]
    x = x3.reshape(A_TILE * SEQ_PAD, D_MODEL)
    m = jnp.full((D_MODEL, D_MODEL), 1.0 / D_MODEL, dtype=jnp.float32)
    mean = jnp.dot(x, m, preferred_element_type=jnp.float32)
    ex2 = jnp.dot(x * x, m, preferred_element_type=jnp.float32)
    var = ex2 - mean * mean
    normed = (x - mean) * jax.lax.rsqrt(var + EPS)
    out_ref[...] = normed * w_ref[...] + b_ref[...]


def _pid_kernel(acts_ref, out_ref):
    c = jax.lax.broadcasted_iota(jnp.int32, (PID_TILE, 2 * SEQ), 1)
    s = jnp.where(c < SEQ, c, c - SEQ)
    out_ref[...] = acts_ref[...] * SEQ_PAD + s


NIDX = 8
PAIRS_PW = ROWS_PW // 2
CHUNK = 2 * SEQ


def _sc_gather_kernel(tab_hbm, pid_hbm, out_hbm, idx, rows, si, sg, sw):
    wid = lax.axis_index("s") * 2 + lax.axis_index("c")
    pr0 = wid * PAIRS_PW

    for k in range(4):
        pltpu.async_copy(pid_hbm.at[pr0 + k], idx[k], si[k])

    @pl.loop(0, PAIRS_PW, step=NIDX)
    def _(r):
        for k in range(NIDX):
            rr = r + k
            j = k % 2
            b0 = (pr0 + rr) * 2

            @pl.when(rr >= 2)
            def _():
                pltpu.make_async_copy(
                    rows[j].at[pl.ds(0, SEQ)], out_hbm.at[b0 - 4], sw[j]
                ).wait()
                pltpu.make_async_copy(
                    rows[j].at[pl.ds(SEQ, SEQ)], out_hbm.at[b0 - 3], sw[j]
                ).wait()

            pltpu.make_async_copy(pid_hbm.at[pr0 + rr], idx[k], si[k]).wait()
            pltpu.async_copy(tab_hbm.at[idx[k]], rows[j], sg[j])
            pltpu.make_async_copy(tab_hbm.at[idx[k]], rows[j], sg[j]).wait()

            kf = (k + 4) % NIDX

            @pl.when(rr + 4 < PAIRS_PW)
            def _():
                pltpu.async_copy(pid_hbm.at[pr0 + rr + 4], idx[kf], si[kf])

            pltpu.async_copy(
                rows[j].at[pl.ds(0, SEQ)], out_hbm.at[b0], sw[j]
            )
            pltpu.async_copy(
                rows[j].at[pl.ds(SEQ, SEQ)], out_hbm.at[b0 + 1], sw[j]
            )

    for j in range(2):
        rr = PAIRS_PW - 2 + j
        b0 = (pr0 + rr) * 2
        pltpu.make_async_copy(
            rows[j].at[pl.ds(0, SEQ)], out_hbm.at[b0], sw[j]
        ).wait()
        pltpu.make_async_copy(
            rows[j].at[pl.ds(SEQ, SEQ)], out_hbm.at[b0 + 1], sw[j]
        ).wait()


def _sc_gather_body(tab_hbm, pid_hbm, out_hbm, *rest):
    idx = list(rest[:NIDX])
    rows = list(rest[NIDX : NIDX + 2])
    si = list(rest[NIDX + 2 : 2 * NIDX + 2])
    sg = list(rest[2 * NIDX + 2 : 2 * NIDX + 4])
    sw = list(rest[2 * NIDX + 4 : 2 * NIDX + 6])
    _sc_gather_kernel(tab_hbm, pid_hbm, out_hbm, idx, rows, si, sg, sw)


@jax.jit
def kernel(actions, att_mask, action_table, pos_table, ln_weight, ln_bias):
    batch = actions.shape[0]
    cls_col = jnp.full((batch, 1), NUM_ACTIONS, dtype=actions.dtype)
    acts = jnp.concatenate([cls_col, actions], axis=1)
    pos_pad = jnp.pad(pos_table, ((0, SEQ_PAD - SEQ), (0, 0)))
    mask = jnp.concatenate(
        [jnp.zeros((batch, 1), dtype=att_mask.dtype), att_mask], axis=1
    )

    tab_flat = pl.pallas_call(
        _pairtab_kernel,
        grid=(pl.cdiv(VOCAB, A_TILE),),
        in_specs=[
            pl.BlockSpec((A_TILE, D_MODEL), lambda i: (i, 0)),
            pl.BlockSpec((SEQ_PAD, D_MODEL), lambda i: (0, 0)),
            pl.BlockSpec((1, D_MODEL), lambda i: (0, 0)),
            pl.BlockSpec((1, D_MODEL), lambda i: (0, 0)),
        ],
        out_specs=pl.BlockSpec((A_TILE * SEQ_PAD, D_MODEL), lambda i: (i, 0)),
        out_shape=jax.ShapeDtypeStruct((VOCAB * SEQ_PAD, D_MODEL), jnp.float32),
    )(
        action_table,
        pos_pad,
        ln_weight.reshape(1, D_MODEL),
        ln_bias.reshape(1, D_MODEL),
    )

    acts2 = acts.reshape(batch // 2, 2 * SEQ)
    pid = pl.pallas_call(
        _pid_kernel,
        grid=(batch // 2 // PID_TILE,),
        in_specs=[pl.BlockSpec((PID_TILE, 2 * SEQ), lambda i: (i, 0))],
        out_specs=pl.BlockSpec((PID_TILE, 2 * SEQ), lambda i: (i, 0)),
        out_shape=jax.ShapeDtypeStruct((batch // 2, 2 * SEQ), jnp.int32),
    )(acts2)

    mesh = plsc.VectorSubcoreMesh(core_axis_name="c", subcore_axis_name="s")
    sc_gather = functools.partial(
        pl.kernel,
        mesh=mesh,
        out_type=jax.ShapeDtypeStruct((batch, SEQ, D_MODEL), jnp.float32),
        scratch_types=(
            [pltpu.VMEM((CHUNK,), jnp.int32)] * NIDX
            + [pltpu.VMEM((CHUNK, D_MODEL), jnp.float32)] * 2
            + [pltpu.SemaphoreType.DMA] * (NIDX + 4)
        ),
    )(_sc_gather_body)
    out = sc_gather(tab_flat, pid)

    return (out, mask)
